# baseline (speedup 1.0000x reference)
import functools
import jax
import jax.numpy as jnp
from jax import lax
from jax.experimental import pallas as pl
from jax.experimental.pallas import tpu as pltpu
from jax.experimental.pallas import tpu_sc as plsc

_CAP = 1000000
_N = 16384
_D = 64
_C = 256
_GSC = 999424 // _C
_CREM = _CAP % _C
_TBLK0 = 1024
_NWORK = 32
_NBUF = 6
_ITERS = _GSC // _NWORK
_TT = _ITERS // _NBUF + 2
_FW = _N + 2 * _TBLK0 + _C
_TBLK = _TBLK0
_NBLK = (_CAP + _TBLK - 1) // _TBLK
_TGRID = _N // _TBLK + 3


def _sc_body(q_hbm, o_hbm, vq, in_sem, out_sem):
    wid = lax.axis_index("s") * 2 + lax.axis_index("c")

    def g_of(i):
        return wid * _ITERS + i

    def pipe_step(t, _):
        for k in range(_NBUF):
            i = t * _NBUF + k

            @pl.when(jnp.logical_and(i >= _NBUF, i - _NBUF < _ITERS))
            def _drain_out():
                sp = pl.multiple_of(g_of(i - _NBUF) * _C, 128)
                pltpu.make_async_copy(
                    vq.at[k], o_hbm.at[:, pl.ds(sp, _C)], out_sem.at[k]).wait()

            @pl.when(i < _ITERS)
            def _start_in():
                s = pl.multiple_of(g_of(i) * _C, 128)
                pltpu.make_async_copy(
                    q_hbm.at[:, pl.ds(s, _C)], vq.at[k], in_sem.at[k]).start()

            kp = (k + _NBUF - 1) % _NBUF

            @pl.when(jnp.logical_and(i >= 1, i - 1 < _ITERS))
            def _flip_prev():
                sp = pl.multiple_of(g_of(i - 1) * _C, 128)
                pltpu.make_async_copy(
                    q_hbm.at[:, pl.ds(sp, _C)], vq.at[kp], in_sem.at[kp]).wait()
                pltpu.make_async_copy(
                    vq.at[kp], o_hbm.at[:, pl.ds(sp, _C)], out_sem.at[kp]).start()
        return 0

    lax.fori_loop(0, _TT, pipe_step, 0)


@functools.cache
def _sc_call():
    mesh = plsc.VectorSubcoreMesh(
        core_axis_name="c", subcore_axis_name="s",
        num_cores=2, num_subcores=16)
    return functools.partial(
        pl.kernel,
        out_type=jax.ShapeDtypeStruct((_D, _CAP), jnp.float32),
        mesh=mesh,
        scratch_types=[
            pltpu.VMEM((_NBUF, _D, _C), jnp.float32),
            pltpu.SemaphoreType.DMA((_NBUF,)),
            pltpu.SemaphoreType.DMA((_NBUF,)),
        ],
    )(_sc_body)


def _tc_blk(t, sp):
    p0b = sp[0] // _TBLK
    b = p0b + t
    b = jnp.where(b >= _NBLK, b - _NBLK, b)
    return jnp.where(t == _TGRID - 1, _NBLK - 1, b)


def _tc_merge_body(sp_ref, q_ref, f1_ref, f2_ref, sc_ref, o_ref):
    del sc_ref
    ptr = sp_ref[0]
    t = pl.program_id(0)
    blk = _tc_blk(t, sp_ref)
    s = blk * _TBLK
    col = s + lax.broadcasted_iota(jnp.int32, (1, _TBLK), 1)
    off = col - ptr
    off = jnp.where(off < 0, off + _CAP, off)
    mask = off < _N
    a = ptr % _C
    a2 = jnp.where(a >= _CREM, a - _CREM, a + _C - _CREM)
    usef2 = s < ptr + _N - _CAP
    fs1 = pl.multiple_of(jnp.clip(s + a + _TBLK - ptr, 0, _FW - _TBLK), 128)
    fs2 = pl.multiple_of(
        jnp.clip(s + _CAP - ptr + a2, 0, _FW - _TBLK), 128)
    fblk = jnp.where(usef2, f2_ref[:, pl.ds(fs2, _TBLK)],
                     f1_ref[:, pl.ds(fs1, _TBLK)])
    o_ref[...] = jnp.where(mask, fblk, q_ref[...])


def _tc_merge(qt, f1, f2, out_sc, pvec):
    grid_spec = pltpu.PrefetchScalarGridSpec(
        num_scalar_prefetch=1,
        grid=(_TGRID,),
        in_specs=[
            pl.BlockSpec((_D, _TBLK), lambda t, sp: (0, _tc_blk(t, sp))),
            pl.BlockSpec((_D, _FW), lambda t, sp: (0, 0)),
            pl.BlockSpec((_D, _FW), lambda t, sp: (0, 0)),
            pl.BlockSpec(memory_space=pltpu.MemorySpace.HBM),
        ],
        out_specs=pl.BlockSpec((_D, _TBLK), lambda t, sp: (0, _tc_blk(t, sp))),
    )
    return pl.pallas_call(
        _tc_merge_body,
        grid_spec=grid_spec,
        out_shape=jax.ShapeDtypeStruct((_D, _CAP), jnp.float32),
        input_output_aliases={4: 0},
    )(pvec[:1], qt, f1, f2, out_sc)


def kernel(queue, features, ptr):
    ptr = jnp.asarray(ptr, jnp.int32)
    qt = jnp.swapaxes(queue, 0, 1)
    ft = jnp.swapaxes(features, 0, 1)
    a = ptr % _C
    a2 = jnp.where(a >= _CREM, a - _CREM, a + _C - _CREM)
    f1 = jax.lax.dynamic_update_slice(
        jnp.zeros((_D, _FW), jnp.float32), ft, (0, a + _TBLK))
    f2 = jax.lax.dynamic_update_slice(
        jnp.zeros((_D, _FW), jnp.float32), ft, (0, a2))
    pvec = ptr.reshape(1)
    out_sc = _sc_call()(qt)
    out_t = _tc_merge(qt, f1, f2, out_sc, pvec)
    return jnp.swapaxes(out_t, 0, 1)
